# Initial kernel scaffold; baseline (speedup 1.0000x reference)
import jax
import jax.numpy as jnp
from jax.experimental import pallas as pl


def kernel(x):
    raise NotImplementedError("write your pallas kernel here")


# speedup vs baseline: 2.4721x; 2.4721x over previous
import functools

import numpy as np
import jax
import jax.numpy as jnp
from jax import lax
from jax.experimental import pallas as pl
from jax.experimental.pallas import tpu as pltpu
from jax.experimental.pallas import tpu_sc as plsc

_NC, _NS, _L = 2, 16, 16
_NW = _NC * _NS
_P = 0.15


def _threefry2x32(k0, k1, x0, x1):
    x0 = x0.astype(np.uint32).copy()
    x1 = x1.astype(np.uint32).copy()

    def rotl(v, r):
        return ((v << np.uint32(r)) | (v >> np.uint32(32 - r))).astype(np.uint32)

    ks0 = np.uint32(k0)
    ks1 = np.uint32(k1)
    ks2 = np.uint32(ks0 ^ ks1 ^ np.uint32(0x1BD11BDA))
    ks = (ks0, ks1, ks2)
    x0 = (x0 + ks0).astype(np.uint32)
    x1 = (x1 + ks1).astype(np.uint32)
    r1 = (13, 15, 26, 6)
    r2 = (17, 29, 16, 24)
    for r in range(5):
        for rot in (r1 if r % 2 == 0 else r2):
            x0 = (x0 + x1).astype(np.uint32)
            x1 = rotl(x1, rot) ^ x0
        x0 = (x0 + ks[(r + 1) % 3]).astype(np.uint32)
        x1 = (x1 + ks[(r + 2) % 3] + np.uint32(r + 1)).astype(np.uint32)
    return x0, x1


def _np_bits(k0, k1, n):
    y0, y1 = _threefry2x32(k0, k1, np.zeros(n, np.uint32),
                           np.arange(n, dtype=np.uint32))
    return y0 ^ y1


def _np_split(k0, k1):
    y0, y1 = _threefry2x32(k0, k1, np.zeros(2, np.uint32),
                           np.arange(2, dtype=np.uint32))
    return (y0[0], y1[0]), (y0[1], y1[1])


def _np_uniform(k0, k1, m):
    bits = _np_bits(k0, k1, m)
    fb = (bits >> np.uint32(9)) | np.uint32(0x3F800000)
    return fb.view(np.float32) - np.float32(1.0)


@functools.lru_cache(maxsize=None)
def _swap_plan(b, c):
    n = b * c
    (k10, k11), (k20, k21) = _np_split(0, 42)
    u1 = _np_uniform(k10, k11, n).reshape(b, c)
    u2 = _np_uniform(k20, k21, n).reshape(b, c)
    mask = u1 > np.float32(1.0 - _P)
    rows = np.floor(u2 * np.float32(b)).astype(np.int32)
    delta = (rows.astype(np.int64) * mask.astype(np.int64) * c).reshape(-1)
    src = np.arange(n, dtype=np.int64) + delta
    src = np.where(src >= n, src - n, src).astype(np.int32)

    cs = n // _NW
    swapped = np.nonzero(src != np.arange(n, dtype=np.int32))[0]
    per = [swapped[(swapped >= w * cs) & (swapped < (w + 1) * cs)]
           for w in range(_NW)]
    kmax = max(len(p) for p in per)
    kpad = -(-kmax // _L) * _L
    src_all = np.zeros((_NW, kpad), dtype=np.int32)
    off_all = np.empty((_NW, kpad), dtype=np.int32)
    off_all[:] = cs + (np.arange(kpad, dtype=np.int32) % _L)
    for w, p in enumerate(per):
        src_all[w, :len(p)] = src[p]
        off_all[w, :len(p)] = (p - w * cs).astype(np.int32)
    return cs, kpad, src_all, off_all


@functools.lru_cache(maxsize=None)
def _build(b, c):
    cs, kpad, src_all, off_all = _swap_plan(b, c)
    n = b * c
    mesh = plsc.VectorSubcoreMesh(core_axis_name="c", subcore_axis_name="s",
                                  num_cores=_NC, num_subcores=_NS)

    @functools.partial(
        pl.kernel,
        out_type=jax.ShapeDtypeStruct((n,), jnp.float32),
        mesh=mesh,
        scratch_types=[
            pltpu.VMEM((cs + _L,), jnp.float32),
            pltpu.VMEM((kpad,), jnp.int32),
            pltpu.VMEM((kpad,), jnp.int32),
            pltpu.VMEM((kpad,), jnp.float32),
            pltpu.SemaphoreType.DMA,
            pltpu.SemaphoreType.DMA,
        ],
        compiler_params=pltpu.CompilerParams(needs_layout_passes=False),
    )
    def body(x_hbm, srcs_hbm, offs_hbm, out_hbm,
             chunk_v, src_v, off_v, val_v, sem_c, sem_g):
        wid = lax.axis_index("s") * _NC + lax.axis_index("c")
        base = wid * cs
        cp_in = pltpu.make_async_copy(
            x_hbm.at[pl.ds(base, cs)], chunk_v.at[pl.ds(0, cs)], sem_c)
        cp_in.start()
        pltpu.sync_copy(srcs_hbm.at[wid], src_v)
        gat = pltpu.make_async_copy(x_hbm.at[src_v], val_v, sem_g)
        gat.start()
        pltpu.sync_copy(offs_hbm.at[wid], off_v)
        cp_in.wait()
        gat.wait()

        def fix(j, carry):
            offs = off_v[pl.ds(j * _L, _L)]
            vals = val_v[pl.ds(j * _L, _L)]
            plsc.store_scatter(chunk_v, [offs], vals)
            return carry

        lax.fori_loop(0, kpad // _L, fix, 0)
        pltpu.sync_copy(chunk_v.at[pl.ds(0, cs)],
                        out_hbm.at[pl.ds(base, cs)])

    s_const = jnp.asarray(src_all)
    o_const = jnp.asarray(off_all)

    def run(xf):
        return body(xf, s_const, o_const)

    return run


def kernel(x):
    b, c = x.shape
    out = _build(b, c)(x.reshape(-1).astype(jnp.float32))
    return out.reshape(b, c).astype(x.dtype)


# speedup vs baseline: 2.8733x; 1.1623x over previous
import functools

import numpy as np
import jax
import jax.numpy as jnp
from jax import lax
from jax.experimental import pallas as pl
from jax.experimental.pallas import tpu as pltpu
from jax.experimental.pallas import tpu_sc as plsc

_NC, _NS, _L = 2, 16, 16
_NW = _NC * _NS
_P = 0.15


def _threefry2x32(k0, k1, x0, x1):
    x0 = x0.astype(np.uint32).copy()
    x1 = x1.astype(np.uint32).copy()

    def rotl(v, r):
        return ((v << np.uint32(r)) | (v >> np.uint32(32 - r))).astype(np.uint32)

    ks0 = np.uint32(k0)
    ks1 = np.uint32(k1)
    ks2 = np.uint32(ks0 ^ ks1 ^ np.uint32(0x1BD11BDA))
    ks = (ks0, ks1, ks2)
    x0 = (x0 + ks0).astype(np.uint32)
    x1 = (x1 + ks1).astype(np.uint32)
    r1 = (13, 15, 26, 6)
    r2 = (17, 29, 16, 24)
    for r in range(5):
        for rot in (r1 if r % 2 == 0 else r2):
            x0 = (x0 + x1).astype(np.uint32)
            x1 = rotl(x1, rot) ^ x0
        x0 = (x0 + ks[(r + 1) % 3]).astype(np.uint32)
        x1 = (x1 + ks[(r + 2) % 3] + np.uint32(r + 1)).astype(np.uint32)
    return x0, x1


def _np_bits(k0, k1, n):
    y0, y1 = _threefry2x32(k0, k1, np.zeros(n, np.uint32),
                           np.arange(n, dtype=np.uint32))
    return y0 ^ y1


def _np_split(k0, k1):
    y0, y1 = _threefry2x32(k0, k1, np.zeros(2, np.uint32),
                           np.arange(2, dtype=np.uint32))
    return (y0[0], y1[0]), (y0[1], y1[1])


def _np_uniform(k0, k1, m):
    bits = _np_bits(k0, k1, m)
    fb = (bits >> np.uint32(9)) | np.uint32(0x3F800000)
    return fb.view(np.float32) - np.float32(1.0)


@functools.lru_cache(maxsize=None)
def _swap_plan(b, c):
    n = b * c
    (k10, k11), (k20, k21) = _np_split(0, 42)
    u1 = _np_uniform(k10, k11, n).reshape(b, c)
    u2 = _np_uniform(k20, k21, n).reshape(b, c)
    mask = u1 > np.float32(1.0 - _P)
    rows = np.floor(u2 * np.float32(b)).astype(np.int32)
    delta = (rows.astype(np.int64) * mask.astype(np.int64) * c).reshape(-1)
    src = np.arange(n, dtype=np.int64) + delta
    src = np.where(src >= n, src - n, src).astype(np.int32)

    rows_per = b // _NW
    cs = rows_per * c
    swapped = np.nonzero(src != np.arange(n, dtype=np.int32))[0]
    per = [swapped[(swapped >= w * cs) & (swapped < (w + 1) * cs)]
           for w in range(_NW)]
    kmax = max(len(p) for p in per)
    kpad = -(-kmax // _L) * _L
    src_all = np.zeros((_NW, kpad), dtype=np.int32)
    offr_all = np.empty((_NW, kpad), dtype=np.int32)
    offc_all = np.empty((_NW, kpad), dtype=np.int32)
    offr_all[:] = rows_per
    offc_all[:] = np.arange(kpad, dtype=np.int32) % _L
    for w, p in enumerate(per):
        loc = (p - w * cs).astype(np.int32)
        src_all[w, :len(p)] = src[p]
        offr_all[w, :len(p)] = loc // c
        offc_all[w, :len(p)] = loc % c
    return rows_per, kpad, src_all, offr_all, offc_all


@functools.lru_cache(maxsize=None)
def _build(b, c):
    rows_per, kpad, src_all, offr_all, offc_all = _swap_plan(b, c)
    mesh = plsc.VectorSubcoreMesh(core_axis_name="c", subcore_axis_name="s",
                                  num_cores=_NC, num_subcores=_NS)

    @functools.partial(
        pl.kernel,
        out_type=jax.ShapeDtypeStruct((b, c), jnp.float32),
        mesh=mesh,
        scratch_types=[
            pltpu.VMEM((rows_per + 1, c), jnp.float32),
            pltpu.VMEM((kpad,), jnp.int32),
            pltpu.VMEM((kpad,), jnp.int32),
            pltpu.VMEM((kpad,), jnp.int32),
            pltpu.VMEM((kpad,), jnp.float32),
            pltpu.SemaphoreType.DMA,
            pltpu.SemaphoreType.DMA,
        ],
        compiler_params=pltpu.CompilerParams(needs_layout_passes=False),
    )
    def body(x_hbm, xflat_hbm, srcs_hbm, offr_hbm, offc_hbm, out_hbm,
             chunk_v, src_v, offr_v, offc_v, val_v, sem_c, sem_g):
        wid = lax.axis_index("s") * _NC + lax.axis_index("c")
        row0 = wid * rows_per
        cp_in = pltpu.make_async_copy(
            x_hbm.at[pl.ds(row0, rows_per), :],
            chunk_v.at[pl.ds(0, rows_per), :], sem_c)
        cp_in.start()
        pltpu.sync_copy(srcs_hbm.at[wid], src_v)
        gat = pltpu.make_async_copy(xflat_hbm.at[src_v], val_v, sem_g)
        gat.start()
        pltpu.sync_copy(offr_hbm.at[wid], offr_v)
        pltpu.sync_copy(offc_hbm.at[wid], offc_v)
        cp_in.wait()
        gat.wait()

        def fix(j, carry):
            offr = offr_v[pl.ds(j * _L, _L)]
            offc = offc_v[pl.ds(j * _L, _L)]
            vals = val_v[pl.ds(j * _L, _L)]
            plsc.store_scatter(chunk_v, [offr, offc], vals)
            return carry

        lax.fori_loop(0, kpad // _L, fix, 0)
        pltpu.sync_copy(chunk_v.at[pl.ds(0, rows_per), :],
                        out_hbm.at[pl.ds(row0, rows_per), :])

    s_const = jnp.asarray(src_all)
    r_const = jnp.asarray(offr_all)
    c_const = jnp.asarray(offc_all)

    def run(x, xflat):
        return body(x, xflat, s_const, r_const, c_const)

    return run


def kernel(x):
    b, c = x.shape
    return _build(b, c)(x, x.reshape(-1))


# speedup vs baseline: 2.9650x; 1.0319x over previous
import functools

import numpy as np
import jax
import jax.numpy as jnp
from jax import lax
from jax.experimental import pallas as pl
from jax.experimental.pallas import tpu as pltpu
from jax.experimental.pallas import tpu_sc as plsc

_NC, _NS, _L = 2, 16, 16
_NW = _NC * _NS
_P = 0.15


def _threefry2x32(k0, k1, x0, x1):
    x0 = x0.astype(np.uint32).copy()
    x1 = x1.astype(np.uint32).copy()

    def rotl(v, r):
        return ((v << np.uint32(r)) | (v >> np.uint32(32 - r))).astype(np.uint32)

    ks0 = np.uint32(k0)
    ks1 = np.uint32(k1)
    ks2 = np.uint32(ks0 ^ ks1 ^ np.uint32(0x1BD11BDA))
    ks = (ks0, ks1, ks2)
    x0 = (x0 + ks0).astype(np.uint32)
    x1 = (x1 + ks1).astype(np.uint32)
    r1 = (13, 15, 26, 6)
    r2 = (17, 29, 16, 24)
    for r in range(5):
        for rot in (r1 if r % 2 == 0 else r2):
            x0 = (x0 + x1).astype(np.uint32)
            x1 = rotl(x1, rot) ^ x0
        x0 = (x0 + ks[(r + 1) % 3]).astype(np.uint32)
        x1 = (x1 + ks[(r + 2) % 3] + np.uint32(r + 1)).astype(np.uint32)
    return x0, x1


def _np_bits(k0, k1, n):
    y0, y1 = _threefry2x32(k0, k1, np.zeros(n, np.uint32),
                           np.arange(n, dtype=np.uint32))
    return y0 ^ y1


def _np_split(k0, k1):
    y0, y1 = _threefry2x32(k0, k1, np.zeros(2, np.uint32),
                           np.arange(2, dtype=np.uint32))
    return (y0[0], y1[0]), (y0[1], y1[1])


def _np_uniform(k0, k1, m):
    bits = _np_bits(k0, k1, m)
    fb = (bits >> np.uint32(9)) | np.uint32(0x3F800000)
    return fb.view(np.float32) - np.float32(1.0)


@functools.lru_cache(maxsize=None)
def _swap_plan(b, c):
    n = b * c
    (k10, k11), (k20, k21) = _np_split(0, 42)
    u1 = _np_uniform(k10, k11, n).reshape(b, c)
    u2 = _np_uniform(k20, k21, n).reshape(b, c)
    mask = u1 > np.float32(1.0 - _P)
    rows = np.floor(u2 * np.float32(b)).astype(np.int32)
    delta = (rows.astype(np.int64) * mask.astype(np.int64) * c).reshape(-1)
    src = np.arange(n, dtype=np.int64) + delta
    src = np.where(src >= n, src - n, src).astype(np.int32)

    rows_per = b // _NW
    cs = rows_per * c
    swapped = np.nonzero(src != np.arange(n, dtype=np.int32))[0]
    per = [swapped[(swapped >= w * cs) & (swapped < (w + 1) * cs)]
           for w in range(_NW)]
    kmax = max(len(p) for p in per)
    kpad = -(-kmax // _L) * _L
    src_all = np.zeros((_NW, kpad), dtype=np.int32)
    offr_all = np.empty((_NW, kpad), dtype=np.int32)
    offc_all = np.empty((_NW, kpad), dtype=np.int32)
    offr_all[:] = rows_per
    offc_all[:] = np.arange(kpad, dtype=np.int32) % _L
    for w, p in enumerate(per):
        loc = (p - w * cs).astype(np.int32)
        src_all[w, :len(p)] = src[p]
        offr_all[w, :len(p)] = loc // c
        offc_all[w, :len(p)] = loc % c
    return rows_per, kpad, src_all, offr_all, offc_all


@functools.lru_cache(maxsize=None)
def _build(b, c):
    rows_per, kpad, src_all, offr_all, offc_all = _swap_plan(b, c)
    mesh = plsc.VectorSubcoreMesh(core_axis_name="c", subcore_axis_name="s",
                                  num_cores=_NC, num_subcores=_NS)

    @functools.partial(
        pl.kernel,
        out_type=jax.ShapeDtypeStruct((b, c), jnp.float32),
        mesh=mesh,
        scratch_types=[
            pltpu.VMEM((rows_per + 1, c), jnp.float32),
            pltpu.VMEM((kpad,), jnp.int32),
            pltpu.VMEM((kpad,), jnp.int32),
            pltpu.VMEM((kpad,), jnp.int32),
            pltpu.VMEM((kpad,), jnp.float32),
            pltpu.SemaphoreType.DMA,
            pltpu.SemaphoreType.DMA,
        ],
        compiler_params=pltpu.CompilerParams(needs_layout_passes=False),
    )
    def body(x_hbm, xflat_hbm, srcs_hbm, offr_hbm, offc_hbm, out_hbm,
             chunk_v, src_v, offr_v, offc_v, val_v, sem_c, sem_g):
        wid = lax.axis_index("s") * _NC + lax.axis_index("c")
        row0 = wid * rows_per
        cp_in = pltpu.make_async_copy(
            x_hbm.at[pl.ds(row0, rows_per), :],
            chunk_v.at[pl.ds(0, rows_per), :], sem_c)
        cp_in.start()
        pltpu.sync_copy(srcs_hbm.at[wid], src_v)
        gat = pltpu.make_async_copy(xflat_hbm.at[src_v], val_v, sem_g)
        gat.start()
        pltpu.sync_copy(offr_hbm.at[wid], offr_v)
        pltpu.sync_copy(offc_hbm.at[wid], offc_v)
        cp_in.wait()
        gat.wait()

        @plsc.parallel_loop(0, kpad, step=_L, unroll=8)
        def fix(i):
            offr = offr_v[pl.ds(i, _L)]
            offc = offc_v[pl.ds(i, _L)]
            vals = val_v[pl.ds(i, _L)]
            plsc.store_scatter(chunk_v, [offr, offc], vals)
        pltpu.sync_copy(chunk_v.at[pl.ds(0, rows_per), :],
                        out_hbm.at[pl.ds(row0, rows_per), :])

    s_const = jnp.asarray(src_all)
    r_const = jnp.asarray(offr_all)
    c_const = jnp.asarray(offc_all)

    def run(x, xflat):
        return body(x, xflat, s_const, r_const, c_const)

    return run


def kernel(x):
    b, c = x.shape
    return _build(b, c)(x, x.reshape(-1))
